# baseline (speedup 1.0000x reference)
import jax
import jax.numpy as jnp
import numpy as np
from jax import lax
from jax.experimental import pallas as pl
from jax.experimental.pallas import tpu as pltpu
from jax.experimental.pallas import tpu_sc as plsc

N = 1600000
NB = N // 128
V1, D1 = 100000, 16
V8, D8 = 12500, 20
NW = 32
VPW1 = 3128
V1PAD = NW * VPW1
VPW8 = 400
V8PAD = NW * VPW8
W1 = D1 + 1
W8 = D8 + 1
A1 = (((VPW1 + 1) * W1 + 15) // 16) * 16
A8 = (((VPW8 + 1) * W8 + 15) // 16) * 16
CH = 1024
CHB = CH // 128
P1 = 184
P8 = VPW8
SEARCH_ITERS = 21


def _lower_bound(ids_hbm, probe_v, target, iota):
    def step(_, lohi):
        lo, hi = lohi
        mid = jnp.minimum((lo + hi) // 2, N - 1)
        m8 = jnp.minimum((mid // 8) * 8, N - 16)
        pltpu.sync_copy(ids_hbm.at[pl.ds(m8, 16)], probe_v)
        pv = probe_v[...]
        val = jnp.max(jnp.where(iota == mid - m8, pv,
                                np.int32(-2147483648)))
        big = val >= target
        return (jnp.where(big, lo, mid + 1), jnp.where(big, mid, hi))
    lo, _ = lax.fori_loop(0, SEARCH_ITERS,
                          step, (np.int32(0), np.int32(N)))
    return lo


def _do_scale(data_hbm, ids_hbm, out_hbm, acc_v, data_v, ids_v, stage_v,
              probe_v, wid, vpw, d, w, ap, piece, ntr):
    iota = lax.iota(jnp.int32, 16)
    v0 = wid * vpw
    r0 = _lower_bound(ids_hbm, probe_v, v0, iota)
    r1 = _lower_bound(ids_hbm, probe_v, v0 + vpw, iota)

    zero16 = (iota * 0).astype(jnp.float32)

    def zstep(i, c):
        acc_v[pl.ds(i * 16, 16)] = zero16
        return c
    lax.fori_loop(0, ap // 16, zstep, 0)

    m_lane0 = iota == 0
    m_hi4 = iota >= 12
    ones = zero16 + 1.0
    uvpw = np.uint32(vpw)
    tr_v = iota // 8
    f_v = iota - (iota // 8) * 8
    tr2_v = iota * 0 + 2
    f2_v = iota - 12

    a0 = (r0 // 128) * 128
    nch = (r1 - a0 + CH - 1) // CH

    def chunk_step(k, c):
        g = a0 + k * CH
        s = jnp.minimum(g, N - CH)
        goff = g - s
        for t in range(ntr):
            pltpu.sync_copy(data_hbm.at[t, pl.ds(s // 128, CHB)],
                            data_v.at[t])
        pltpu.sync_copy(ids_hbm.at[pl.ds(s, CH)], ids_v)
        ngroups = (CH - goff) // 16

        def group_step(gi, cc):
            j0 = goff + gi * 16
            idvec = ids_v[pl.ds(j0, 16)]
            rel_u = (idvec - v0).astype(jnp.uint32)
            bases = (jnp.minimum(rel_u, uvpw) * w).astype(jnp.int32)
            tcb_v = jnp.full((16,), j0 // 128, jnp.int32)
            p_base = jnp.full((16,), j0 - (j0 // 128) * 128, jnp.int32)
            for kk in range(16):
                b = bases[kk]
                p_v = p_base + kk
                row = plsc.load_gather(data_v, [tr_v, tcb_v, f_v, p_v])
                plsc.addupdate_scatter(acc_v, [b + iota], row)
                if d == 20:
                    row2 = plsc.load_gather(data_v,
                                            [tr2_v, tcb_v, f2_v, p_v],
                                            mask=m_hi4)
                    plsc.addupdate_scatter(acc_v, [b + (iota + 4)], row2,
                                           mask=m_hi4)
                idxc = jnp.full((16,), b + d, jnp.int32)
                plsc.addupdate_scatter(acc_v, [idxc], ones, mask=m_lane0)
            return cc
        lax.fori_loop(0, ngroups, group_step, 0)
        return c
    lax.fori_loop(0, nch, chunk_step, 0)

    onef = (iota * 0 + 1).astype(jnp.float32)

    def piece_step(p, c):
        def vstep(v, cc):
            b = (p * piece + v) * w
            sums = acc_v[pl.ds(b, 16)]
            cntv = acc_v[pl.ds(b + d - 15, 16)]
            cb = jnp.full((16,), cntv[15], jnp.float32)
            inv = onef / jnp.maximum(cb, onef)
            stage_v[v, pl.ds(0, 16)] = sums * inv
            if d == 20:
                sums2 = acc_v[pl.ds(b + 4, 16)]
                stage_v[v, pl.ds(4, 16)] = sums2 * inv
            return cc
        lax.fori_loop(0, piece, vstep, 0)
        pltpu.sync_copy(stage_v, out_hbm.at[pl.ds(v0 + p * piece, piece)])
        return c
    lax.fori_loop(0, vpw // piece, piece_step, 0)


def _body(pts_hbm, lbl_hbm, id1_hbm, id8_hbm, out1_hbm, out8_hbm,
          acc1_v, acc8_v, data1_v, data8_v, ids_v, stage1_v, stage8_v,
          probe_v):
    wid = lax.axis_index("s") * 2 + lax.axis_index("c")
    _do_scale(pts_hbm, id1_hbm, out1_hbm, acc1_v, data1_v, ids_v, stage1_v,
              probe_v, wid, VPW1, D1, W1, A1, P1, 2)
    _do_scale(lbl_hbm, id8_hbm, out8_hbm, acc8_v, data8_v, ids_v, stage8_v,
              probe_v, wid, VPW8, D8, W8, A8, P8, 3)


_mesh = plsc.VectorSubcoreMesh(core_axis_name="c", subcore_axis_name="s",
                               num_cores=2, num_subcores=16)

_run = pl.kernel(
    _body,
    out_type=(jax.ShapeDtypeStruct((V1PAD, D1), jnp.float32),
              jax.ShapeDtypeStruct((V8PAD, D8), jnp.float32)),
    mesh=_mesh,
    compiler_params=pltpu.CompilerParams(needs_layout_passes=False,
                                        use_tc_tiling_on_sc=False),
    scratch_types=[
        pltpu.VMEM((A1,), jnp.float32),
        pltpu.VMEM((A8,), jnp.float32),
        pltpu.VMEM((2, CHB, 8, 128), jnp.float32),
        pltpu.VMEM((3, CHB, 8, 128), jnp.float32),
        pltpu.VMEM((CH,), jnp.int32),
        pltpu.VMEM((P1, D1), jnp.float32),
        pltpu.VMEM((P8, D8), jnp.float32),
        pltpu.VMEM((16,), jnp.int32),
    ],
)


def kernel(points, labels, coors_inv_1, coors_inv_8):
    pts_v = points.reshape(NB, 128, 2, 8).transpose(2, 0, 3, 1)
    lbl24 = jnp.concatenate(
        [labels, jnp.zeros((N, 4), jnp.float32)], axis=1)
    lbl_v = lbl24.reshape(NB, 128, 3, 8).transpose(2, 0, 3, 1)
    o1, o8 = _run(pts_v, lbl_v,
                  coors_inv_1.astype(jnp.int32),
                  coors_inv_8.astype(jnp.int32))
    return o1[:V1, :], o8[:V8, :]


# speedup vs baseline: 1.0556x; 1.0556x over previous
import jax
import jax.numpy as jnp
import numpy as np
from jax import lax
from jax.experimental import pallas as pl
from jax.experimental.pallas import tpu as pltpu
from jax.experimental.pallas import tpu_sc as plsc

N = 1600000
NB = N // 128
V1, D1 = 100000, 16
V8, D8 = 12500, 20
NW = 32
VPW1 = 3128
V1PAD = NW * VPW1
VPW8 = 400
V8PAD = NW * VPW8
W1 = D1 + 1
W8 = D8 + 1
A1 = (((VPW1 + 1) * W1 + 15) // 16) * 16
A8 = (((VPW8 + 1) * W8 + 15) // 16) * 16
CH = 1024
CHB = CH // 128
P1 = 184
P8 = VPW8
SEARCH_ITERS = 21


def _lower_bound(ids_hbm, probe_v, target, iota):
    def step(_, lohi):
        lo, hi = lohi
        mid = jnp.minimum((lo + hi) // 2, N - 1)
        m8 = jnp.minimum((mid // 8) * 8, N - 16)
        pltpu.sync_copy(ids_hbm.at[pl.ds(m8, 16)], probe_v)
        pv = probe_v[...]
        val = jnp.max(jnp.where(iota == mid - m8, pv,
                                np.int32(-2147483648)))
        big = val >= target
        return (jnp.where(big, lo, mid + 1), jnp.where(big, mid, hi))
    lo, _ = lax.fori_loop(0, SEARCH_ITERS,
                          step, (np.int32(0), np.int32(N)))
    return lo


def _do_scale(data_hbm, ids_hbm, out_hbm, acc_v, data_v, ids_v, stage_v,
              probe_v, wid, vpw, d, w, ap, piece, ntr):
    iota = lax.iota(jnp.int32, 16)
    v0 = wid * vpw
    r0 = _lower_bound(ids_hbm, probe_v, v0, iota)
    r1 = _lower_bound(ids_hbm, probe_v, v0 + vpw, iota)

    zero16 = (iota * 0).astype(jnp.float32)

    def zstep(i, c):
        acc_v[pl.ds(i * 16, 16)] = zero16
        return c
    lax.fori_loop(0, ap // 16, zstep, 0)

    m_hi4 = iota >= 12
    uvpw = np.uint32(vpw)

    a0 = (r0 // 128) * 128
    nch = (r1 - a0 + CH - 1) // CH

    patt = (iota // 8) * (CHB * 128 * 8) + (iota - (iota // 8) * 8) * 128
    patt2 = 2 * (CHB * 128 * 8) + (iota - 12) * 128
    iota4 = iota + 4

    def chunk_step(k, c):
        g = a0 + k * CH
        s = jnp.minimum(g, N - CH)
        goff = g - s
        sc = (s // 128) * 1024
        for t in range(ntr):
            pltpu.sync_copy(
                data_hbm.at[pl.ds(t * (NB * 1024) + sc, CHB * 1024)],
                data_v.at[pl.ds(t * (CHB * 1024), CHB * 1024)])
        pltpu.sync_copy(ids_hbm.at[pl.ds(s, CH)], ids_v)
        ngroups = (CH - goff) // 16

        def group_step(gi, cc):
            j0 = goff + gi * 16
            idvec = ids_v[pl.ds(j0, 16)]
            rel_u = (idvec - v0).astype(jnp.uint32)
            bases = (jnp.minimum(rel_u, uvpw) * w).astype(jnp.int32)
            base_g = (j0 // 128) * 1024 + (j0 - (j0 // 128) * 128)
            gb = patt + base_g
            if d == 20:
                gb2 = patt2 + base_g
            for kk in range(16):
                b = bases[kk]
                row = plsc.load_gather(data_v, [gb + kk])
                plsc.addupdate_scatter(acc_v, [b + iota], row)
                if d == 20:
                    row2 = plsc.load_gather(data_v, [gb2 + kk],
                                            mask=m_hi4)
                    plsc.addupdate_scatter(acc_v, [b + iota4], row2,
                                           mask=m_hi4)
            occ, last = plsc.scan_count(bases)
            cntf = occ.astype(jnp.float32)
            plsc.addupdate_scatter(acc_v, [bases + d], cntf, mask=last)
            return cc
        lax.fori_loop(0, ngroups, group_step, 0)
        return c
    lax.fori_loop(0, nch, chunk_step, 0)

    onef = (iota * 0 + 1).astype(jnp.float32)

    def piece_step(p, c):
        def vstep(v, cc):
            b = (p * piece + v) * w
            sums = acc_v[pl.ds(b, 16)]
            cntv = acc_v[pl.ds(b + d - 15, 16)]
            cb = jnp.full((16,), cntv[15], jnp.float32)
            inv = onef / jnp.maximum(cb, onef)
            stage_v[v, pl.ds(0, 16)] = sums * inv
            if d == 20:
                sums2 = acc_v[pl.ds(b + 4, 16)]
                stage_v[v, pl.ds(4, 16)] = sums2 * inv
            return cc
        lax.fori_loop(0, piece, vstep, 0)
        pltpu.sync_copy(stage_v, out_hbm.at[pl.ds(v0 + p * piece, piece)])
        return c
    lax.fori_loop(0, vpw // piece, piece_step, 0)


def _body(pts_hbm, lbl_hbm, id1_hbm, id8_hbm, out1_hbm, out8_hbm,
          acc1_v, acc8_v, data1_v, data8_v, ids_v, stage1_v, stage8_v,
          probe_v):
    wid = lax.axis_index("s") * 2 + lax.axis_index("c")
    _do_scale(pts_hbm, id1_hbm, out1_hbm, acc1_v, data1_v, ids_v, stage1_v,
              probe_v, wid, VPW1, D1, W1, A1, P1, 2)
    _do_scale(lbl_hbm, id8_hbm, out8_hbm, acc8_v, data8_v, ids_v, stage8_v,
              probe_v, wid, VPW8, D8, W8, A8, P8, 3)


_mesh = plsc.VectorSubcoreMesh(core_axis_name="c", subcore_axis_name="s",
                               num_cores=2, num_subcores=16)

_run = pl.kernel(
    _body,
    out_type=(jax.ShapeDtypeStruct((V1PAD, D1), jnp.float32),
              jax.ShapeDtypeStruct((V8PAD, D8), jnp.float32)),
    mesh=_mesh,
    compiler_params=pltpu.CompilerParams(needs_layout_passes=False,
                                        use_tc_tiling_on_sc=False),
    scratch_types=[
        pltpu.VMEM((A1,), jnp.float32),
        pltpu.VMEM((A8,), jnp.float32),
        pltpu.VMEM((2 * CHB * 1024,), jnp.float32),
        pltpu.VMEM((3 * CHB * 1024,), jnp.float32),
        pltpu.VMEM((CH,), jnp.int32),
        pltpu.VMEM((P1, D1), jnp.float32),
        pltpu.VMEM((P8, D8), jnp.float32),
        pltpu.VMEM((16,), jnp.int32),
    ],
)


def kernel(points, labels, coors_inv_1, coors_inv_8):
    pts_v = points.reshape(NB, 128, 2, 8).transpose(2, 0, 3, 1).reshape(-1)
    lbl24 = jnp.concatenate(
        [labels, jnp.zeros((N, 4), jnp.float32)], axis=1)
    lbl_v = lbl24.reshape(NB, 128, 3, 8).transpose(2, 0, 3, 1).reshape(-1)
    o1, o8 = _run(pts_v, lbl_v,
                  coors_inv_1.astype(jnp.int32),
                  coors_inv_8.astype(jnp.int32))
    return o1[:V1, :], o8[:V8, :]
